# baseline (speedup 1.0000x reference)
import jax
import jax.numpy as jnp
from jax.experimental import pallas as pl
from jax.experimental.pallas import tpu as pltpu

_G = 62


def _body(bidx_ref, fm_ref, mask_ref, out_ref, fmp_ref):
    g = pl.program_id(1)
    neg = jnp.finfo(jnp.float32).min

    @pl.when(g == 0)
    def _repack():
        f = fm_ref[:, 0]
        half = f.shape[2] // 2
        fmp_ref[...] = jnp.concatenate(
            [f[:, :, :half, :], f[:, :, half:, :]], axis=3
        )

    results = []
    for j in range(_G):
        b = bidx_ref[g * _G + j]
        f = fmp_ref[b]
        mn = mask_ref[0, j]
        half = mn.shape[0] // 2
        m = jnp.concatenate([mn[:half, :], mn[half:, :]], axis=1)
        masked = jnp.where((m != 0)[None, :, :], f, neg)
        results.append(jnp.max(masked, axis=(1, 2)))
    out_ref[0, 0] = jnp.stack(results)


def kernel(feature_maps, cell_masks, cell_counts):
    B, C, H, W = feature_maps.shape
    n_cells = cell_masks.shape[0]
    C_SPLIT = 8
    C_BLK = C // C_SPLIT
    n_groups = n_cells // _G

    fm = feature_maps.reshape(B, C_SPLIT, C_BLK, H, W)
    masks = cell_masks.reshape(n_groups, _G, H, W).astype(jnp.int8)

    ends = jnp.cumsum(cell_counts)
    batch_idx = jnp.sum(
        jnp.arange(n_cells, dtype=ends.dtype)[:, None] >= ends[None, :],
        axis=1,
        dtype=jnp.int32,
    )
    batch_idx = jnp.minimum(batch_idx, B - 1)

    grid_spec = pltpu.PrefetchScalarGridSpec(
        num_scalar_prefetch=1,
        grid=(C_SPLIT, n_groups),
        in_specs=[
            pl.BlockSpec(
                (B, 1, C_BLK, H, W),
                lambda c, g, bidx: (0, c, 0, 0, 0),
            ),
            pl.BlockSpec(
                (1, _G, H, W),
                lambda c, g, bidx: (g, 0, 0, 0),
            ),
        ],
        out_specs=pl.BlockSpec(
            (1, 1, _G, C_BLK), lambda c, g, bidx: (c, g, 0, 0)
        ),
        scratch_shapes=[
            pltpu.VMEM((B, C_BLK, H // 2, 2 * W), jnp.float32),
        ],
    )

    out = pl.pallas_call(
        _body,
        grid_spec=grid_spec,
        out_shape=jax.ShapeDtypeStruct(
            (C_SPLIT, n_groups, _G, C_BLK), feature_maps.dtype
        ),
    )(batch_idx, fm, masks)
    return out.transpose(1, 2, 0, 3).reshape(n_cells, C)


# speedup vs baseline: 2.0707x; 2.0707x over previous
import jax
import jax.numpy as jnp
from jax.experimental import pallas as pl
from jax.experimental.pallas import tpu as pltpu

_G = 62


def _body(bidx_ref, fm_ref, mask_ref, out_ref, fmp_ref):
    g = pl.program_id(1)
    neg = jnp.finfo(jnp.float32).min

    @pl.when(g == 0)
    def _repack():
        f = fm_ref[:, 0]
        half = f.shape[2] // 2
        fmp_ref[...] = jnp.concatenate(
            [f[:, :, :half, :], f[:, :, half:, :]], axis=3
        )

    results = []
    for j in range(_G):
        b = bidx_ref[g * _G + j]
        f = fmp_ref[b]
        m = mask_ref[0, j]
        masked = jnp.where((m != 0)[None, :, :], f, neg)
        results.append(jnp.max(masked, axis=(1, 2)))
    out_ref[0, 0] = jnp.stack(results)


def kernel(feature_maps, cell_masks, cell_counts):
    B, C, H, W = feature_maps.shape
    n_cells = cell_masks.shape[0]
    C_SPLIT = 8
    C_BLK = C // C_SPLIT
    n_groups = n_cells // _G

    fm = feature_maps.reshape(B, C_SPLIT, C_BLK, H, W)
    cm = cell_masks.astype(jnp.int8)
    masks = jnp.concatenate([cm[:, : H // 2, :], cm[:, H // 2 :, :]], axis=2)
    masks = masks.reshape(n_groups, _G, H // 2, 2 * W)

    ends = jnp.cumsum(cell_counts)
    batch_idx = jnp.sum(
        jnp.arange(n_cells, dtype=ends.dtype)[:, None] >= ends[None, :],
        axis=1,
        dtype=jnp.int32,
    )
    batch_idx = jnp.minimum(batch_idx, B - 1)

    grid_spec = pltpu.PrefetchScalarGridSpec(
        num_scalar_prefetch=1,
        grid=(C_SPLIT, n_groups),
        in_specs=[
            pl.BlockSpec(
                (B, 1, C_BLK, H, W),
                lambda c, g, bidx: (0, c, 0, 0, 0),
            ),
            pl.BlockSpec(
                (1, _G, H // 2, 2 * W),
                lambda c, g, bidx: (g, 0, 0, 0),
            ),
        ],
        out_specs=pl.BlockSpec(
            (1, 1, _G, C_BLK), lambda c, g, bidx: (c, g, 0, 0)
        ),
        scratch_shapes=[
            pltpu.VMEM((B, C_BLK, H // 2, 2 * W), jnp.float32),
        ],
    )

    out = pl.pallas_call(
        _body,
        grid_spec=grid_spec,
        out_shape=jax.ShapeDtypeStruct(
            (C_SPLIT, n_groups, _G, C_BLK), feature_maps.dtype
        ),
    )(batch_idx, fm, masks)
    return out.transpose(1, 2, 0, 3).reshape(n_cells, C)
